# baseline (speedup 1.0000x reference)
import functools

import jax
import jax.numpy as jnp
from jax import lax
from jax.experimental import pallas as pl
from jax.experimental.pallas import tpu as pltpu, tpu_sc as plsc

N = 10000
P = 320000
M = 512
NH = 10000

NC = 2
NS = 16
NW = NC * NS

RL = 128
RW = 80
PPW = RL * RW
PP = PPW * NW
NP = 10016
ROWS_PER_TILE = NP // NS
MROWS = 520
TRASH = 1024
ACC_ROWS = M + TRASH + 16

F32 = jnp.float32
I32 = jnp.int32

_mesh = plsc.VectorSubcoreMesh(core_axis_name="c", subcore_axis_name="s")


def _leaky(a):
    return jnp.where(a >= 0, a, 0.1 * a)


@functools.partial(
    pl.kernel,
    out_type=[
        jax.ShapeDtypeStruct((NP,), I32),
        jax.ShapeDtypeStruct((M,), I32),
        jax.ShapeDtypeStruct((M, 4), F32),
    ],
    mesh=_mesh,
    compiler_params=pltpu.CompilerParams(needs_layout_passes=False, use_tc_tiling_on_sc=False),
    scratch_types=[
        pltpu.VMEM((M,), I32),
        pltpu.VMEM((NP,), I32),
        pltpu.VMEM((M,), I32),
        pltpu.VMEM((M, 4), F32),
        pltpu.VMEM((NP * 2,), F32),
        pltpu.VMEM((NP * 2,), F32),
    ],
)
def _sc_prep(mi_hbm, x_hbm, fp_hbm, nm_hbm, ms_hbm, mg_hbm,
             mib, nmb, msb, mgb, xb, fpb):
    c = lax.axis_index("c")
    s = lax.axis_index("s")

    @pl.when(jnp.logical_and(c == 0, s == 0))
    def _():
        pltpu.sync_copy(mi_hbm, mib)
        pltpu.sync_copy(x_hbm, xb)
        pltpu.sync_copy(fp_hbm, fpb)
        mval = jnp.full((16,), M, I32)

        def fill(i, carry):
            nmb[pl.ds(i * 16, 16)] = mval
            return carry

        lax.fori_loop(0, NP // 16, fill, 0)
        lane = jnp.arange(16, dtype=I32)
        for i in range(M // 16):
            mi = mib[pl.ds(i * 16, 16)]
            plsc.store_scatter(nmb, [mi], i * 16 + lane)
        for i in range(M // 16):
            mi = mib[pl.ds(i * 16, 16)]
            slots = plsc.load_gather(nmb, [mi])
            msb[pl.ds(i * 16, 16)] = slots
            dst = i * 16 + lane
            for col in range(2):
                cc = jnp.full((16,), col, I32)
                plsc.store_scatter(mgb, [dst, cc], plsc.load_gather(fpb, [mi * 2 + col]))
                plsc.store_scatter(mgb, [dst, cc + 2], plsc.load_gather(xb, [mi * 2 + col]))
        pltpu.sync_copy(nmb, nm_hbm)
        pltpu.sync_copy(msb, ms_hbm)
        pltpu.sync_copy(mgb, mg_hbm)


def _tc_xin_body(x_ref, fp_ref, nm_ref, out_ref):
    flag = (nm_ref[...] != M).astype(F32)
    z = jnp.zeros((NP, 1), F32)
    o = jnp.ones((NP, 1), F32)
    out_ref[...] = jnp.concatenate(
        [x_ref[...], fp_ref[...], flag, z, z, z, z, o, z, z, z, z, z, z], axis=1)


_tc_xin = pl.pallas_call(
    _tc_xin_body, out_shape=jax.ShapeDtypeStruct((NP, 16), F32))


PFB = 4096


def _tc_pfw_body(pf_ref, out_ref):
    z1 = jnp.zeros((PFB, 5), F32)
    z2 = jnp.zeros((PFB, 7), F32)
    out_ref[...] = jnp.concatenate([z1, pf_ref[...], z2], axis=1)


_tc_pfw = pl.pallas_call(
    _tc_pfw_body,
    grid=(PP // PFB,),
    in_specs=[pl.BlockSpec((PFB, 4), lambda g: (g, 0))],
    out_specs=pl.BlockSpec((PFB, 16), lambda g: (g, 0)),
    out_shape=jax.ShapeDtypeStruct((PP, 16), F32))


@functools.partial(
    pl.kernel,
    out_type=jax.ShapeDtypeStruct((NC, NP, 16), F32),
    mesh=_mesh,
    compiler_params=pltpu.CompilerParams(needs_layout_passes=False, use_tc_tiling_on_sc=False),
    scratch_types=[
        pltpu.VMEM((RW, RL), I32),
        pltpu.VMEM((RW, RL), I32),
        pltpu.VMEM((RL, 16), F32),
        pltpu.VMEM((RL, 16), F32),
        pltpu.VMEM((RL, 16), F32),
        pltpu.VMEM((RL, 16), F32),
        pltpu.VMEM_SHARED((NP, 16), F32),
        pltpu.SemaphoreType.DMA,
        pltpu.SemaphoreType.DMA,
        pltpu.SemaphoreType.DMA,
        pltpu.SemaphoreType.DMA,
    ],
)
def _sc_stage_a(xin_hbm, pfw_hbm, node_hbm, hedge_hbm, z_hbm, out_hbm,
                nodeb, hedgeb, xr0, xr1, pf0, pf1, accS, sg0, sg1, sp0, sp1):
    c = lax.axis_index("c")
    s = lax.axis_index("s")
    wid = s * NC + c
    rowbase = wid * RW
    pinbase = wid * PPW

    pltpu.sync_copy(z_hbm.at[pl.ds(s * ROWS_PER_TILE, ROWS_PER_TILE)],
                    accS.at[pl.ds(s * ROWS_PER_TILE, ROWS_PER_TILE)])
    pltpu.sync_copy(node_hbm.at[pl.ds(rowbase, RW)], nodeb)
    pltpu.sync_copy(hedge_hbm.at[pl.ds(rowbase, RW)], hedgeb)
    plsc.subcore_barrier()

    def issue(j, xr, pf, sg, sp):
        pltpu.async_copy(xin_hbm.at[nodeb.at[j]], xr, sg)
        pltpu.async_copy(pfw_hbm.at[pl.ds(pinbase + j * RL, RL)], pf, sp)

    def drain(j, xr, pf, sg, sp):
        pltpu.make_async_copy(xin_hbm.at[nodeb.at[j]], xr, sg).wait()
        pltpu.make_async_copy(pfw_hbm.at[pl.ds(pinbase + j * RL, RL)], pf, sp).wait()
        pltpu.sync_copy(xr, accS.at[hedgeb.at[j]], add=True)
        pltpu.sync_copy(pf, accS.at[hedgeb.at[j]], add=True)

    issue(0, xr0, pf0, sg0, sp0)

    def body(jj, carry):
        j0 = jj * 2
        j1 = j0 + 1
        issue(j1, xr1, pf1, sg1, sp1)
        drain(j0, xr0, pf0, sg0, sp0)

        @pl.when(jj < RW // 2 - 1)
        def _():
            issue(j0 + 2, xr0, pf0, sg0, sp0)

        drain(j1, xr1, pf1, sg1, sp1)
        return carry

    lax.fori_loop(0, RW // 2, body, 0)
    plsc.subcore_barrier()
    pltpu.sync_copy(accS.at[pl.ds(s * ROWS_PER_TILE, ROWS_PER_TILE)],
                    out_hbm.at[c, pl.ds(s * ROWS_PER_TILE, ROWS_PER_TILE)])


def _tc_ubar_body(acc_ref, out_ref):
    a = acc_ref[0] + acc_ref[1]
    cnt = a[:, 9:10]
    inv = 1.0 / jnp.maximum(cnt, 1.0)
    t = jnp.where(cnt > 0.5, 1.0, 0.0)
    u = a[:, :9] * inv
    z = jnp.zeros((NP, 1), F32)
    out_ref[...] = jnp.concatenate([u, t, jnp.ones((NP, 1), F32), z, z, z, z, z], axis=1)


_tc_ubar = pl.pallas_call(
    _tc_ubar_body, out_shape=jax.ShapeDtypeStruct((NP, 16), F32))


@functools.partial(
    pl.kernel,
    out_type=jax.ShapeDtypeStruct((NC, MROWS, 16), F32),
    mesh=_mesh,
    compiler_params=pltpu.CompilerParams(needs_layout_passes=False, use_tc_tiling_on_sc=False),
    scratch_types=[
        pltpu.VMEM((NP,), I32),
        pltpu.VMEM((RW, RL), I32),
        pltpu.VMEM((RW, RL), I32),
        pltpu.VMEM((RL, 16), F32),
        pltpu.VMEM((RL, 16), F32),
        pltpu.VMEM((RL,), I32),
        pltpu.VMEM((RL,), I32),
        pltpu.VMEM_SHARED((ACC_ROWS, 16), F32),
        pltpu.SemaphoreType.DMA,
        pltpu.SemaphoreType.DMA,
    ],
)
def _sc_stage_b(ubar_hbm, nm_hbm, node_hbm, hedge_hbm, z_hbm, out_hbm,
                nmv, nodeb, hedgeb, ur0, ur1, sl0, sl1, accS, sg0, sg1):
    c = lax.axis_index("c")
    s = lax.axis_index("s")
    wid = s * NC + c
    rowbase = wid * RW
    zrows = ACC_ROWS // NS

    pltpu.sync_copy(z_hbm.at[pl.ds(s * zrows, zrows)],
                    accS.at[pl.ds(s * zrows, zrows)])
    pltpu.sync_copy(nm_hbm, nmv)
    pltpu.sync_copy(node_hbm.at[pl.ds(rowbase, RW)], nodeb)
    pltpu.sync_copy(hedge_hbm.at[pl.ds(rowbase, RW)], hedgeb)
    plsc.subcore_barrier()

    def issue(j, ur, sg, slotb):
        pltpu.async_copy(ubar_hbm.at[hedgeb.at[j]], ur, sg)

        def inner(i, icarry):
            idx = nodeb[j, pl.ds(i * 16, 16)]
            sl = plsc.load_gather(nmv, [idx])
            slotb[pl.ds(i * 16, 16)] = jnp.where(
                sl == M, M + jnp.bitwise_and(idx, TRASH - 1), sl)
            return icarry

        lax.fori_loop(0, RL // 16, inner, 0)

    def drain(j, ur, sg, slotb):
        pltpu.make_async_copy(ubar_hbm.at[hedgeb.at[j]], ur, sg).wait()
        pltpu.sync_copy(ur, accS.at[slotb], add=True)

    issue(0, ur0, sg0, sl0)

    def body(jj, carry):
        j0 = jj * 2
        j1 = j0 + 1
        issue(j1, ur1, sg1, sl1)
        drain(j0, ur0, sg0, sl0)

        @pl.when(jj < RW // 2 - 1)
        def _():
            issue(j0 + 2, ur0, sg0, sl0)

        drain(j1, ur1, sg1, sl1)
        return carry

    lax.fori_loop(0, RW // 2, body, 0)
    plsc.subcore_barrier()

    @pl.when(s < 13)
    def _():
        pltpu.sync_copy(accS.at[pl.ds(s * 40, 40)],
                        out_hbm.at[c, pl.ds(s * 40, 40)])


def _tc_macro_body(acc_ref, ms_ref, mg_ref,
                   W1_ref, b1_ref, Wp_ref, W2l_ref, b2l_ref, W2r_ref,
                   W3l_ref, b3l_ref, W3r_ref, Wm1_ref, bm1_ref,
                   Wm2_ref, bm2_ref, Wm3_ref, bm3_ref, out_ref):
    hi = lax.Precision.HIGHEST
    acc = acc_ref[0] + acc_ref[1]
    cols = lax.broadcasted_iota(I32, (M, MROWS), 1)
    onehot = (cols == ms_ref[...]).astype(F32)
    H = jnp.dot(onehot, acc, precision=hi)
    U5, U4 = H[:, 0:5], H[:, 5:9]
    T, C = H[:, 9:10], H[:, 10:11]
    pre = (jnp.dot(U5, W1_ref[...], precision=hi) + T * b1_ref[...]
           + jnp.dot(U4, Wp_ref[...], precision=hi))
    hm = _leaky(pre / jnp.maximum(C, 1.0))

    mg = mg_ref[...]
    px, py, sx, sy = mg[:, 0:1], mg[:, 1:2], mg[:, 2:3], mg[:, 3:4]
    ri = lax.broadcasted_iota(I32, (M, M), 0)
    ci = lax.broadcasted_iota(I32, (M, M), 1)
    eye = (ri == ci).astype(F32)
    noteye = 1.0 - eye
    dn = (((0,), (0,)), ((), ()))
    pxr = lax.dot_general(px, eye, dn, precision=hi)
    pyr = lax.dot_general(py, eye, dn, precision=hi)
    adj_v = ((px + sx) <= pxr).astype(F32) * noteye
    adj_h = ((py + sy) <= pyr).astype(F32) * noteye
    ones_col = jnp.ones((M, 1), F32)

    def sage(xf, adj, Wl, bl, Wr):
        deg = lax.dot_general(adj, ones_col, dn, precision=hi)
        agg = lax.dot_general(adj, xf, dn)
        return jnp.dot(agg / jnp.maximum(deg, 1.0), Wl) + bl + jnp.dot(xf, Wr)

    xv = _leaky(sage(hm, adj_v, W2l_ref[...], b2l_ref[...], W2r_ref[...]))
    xh = _leaky(sage(hm, adj_h, W2l_ref[...], b2l_ref[...], W2r_ref[...]))
    xc = jnp.concatenate([xv, xh], axis=1)
    xv2 = _leaky(sage(xc, adj_v, W3l_ref[...], b3l_ref[...], W3r_ref[...]))
    xh2 = _leaky(sage(xc, adj_h, W3l_ref[...], b3l_ref[...], W3r_ref[...]))
    xc2 = jnp.concatenate([xv2, xh2], axis=1)
    o = _leaky(jnp.dot(xc2, Wm1_ref[...]) + bm1_ref[...])
    o = _leaky(jnp.dot(o, Wm2_ref[...]) + bm2_ref[...])
    out_ref[...] = jnp.dot(o, Wm3_ref[...]) + bm3_ref[...]


_tc_macro = pl.pallas_call(
    _tc_macro_body, out_shape=jax.ShapeDtypeStruct((M, 4), F32))


def kernel(x, edge_index, pin_feature, batch, fake_pos, macro_index,
           W1, b1, Wp, W2l, b2l, W2r, W3l, b3l, W3r,
           Wm1, bm1, Wm2, bm2, Wm3, bm3):
    del batch
    node_idx = edge_index[0]
    hedge_idx = edge_index[1]

    pad = PP - P
    prng = jnp.arange(pad, dtype=I32)
    node_pa = jnp.concatenate([node_idx, prng % N]).reshape(NW * RW, RL)
    node_pb = jnp.concatenate([node_idx, N + (prng % 16)]).reshape(NW * RW, RL)
    hedge_p = jnp.concatenate([hedge_idx, NH + (prng % 16)]).reshape(NW * RW, RL)
    pf_p = jnp.concatenate([pin_feature, jnp.zeros((pad, 4), F32)], axis=0)
    x_p = jnp.concatenate([x, jnp.zeros((NP - N, 2), F32)], axis=0)
    fp_p = jnp.concatenate([fake_pos, jnp.zeros((NP - N, 2), F32)], axis=0)
    zeros16 = jnp.zeros((NP, 16), F32)

    nodemap, macroslot, mgeo = _sc_prep(macro_index, x_p.reshape(-1), fp_p.reshape(-1))
    xin16 = _tc_xin(x_p, fp_p, nodemap.reshape(NP, 1))
    pfw = _tc_pfw(pf_p)
    accA = _sc_stage_a(xin16, pfw, node_pa, hedge_p, zeros16)
    ubar = _tc_ubar(accA)
    accB = _sc_stage_b(ubar, nodemap, node_pb, hedge_p, zeros16)
    out = _tc_macro(accB, macroslot.reshape(M, 1), mgeo,
                    W1, b1.reshape(1, 128), Wp, W2l, b2l.reshape(1, 128), W2r,
                    W3l, b3l.reshape(1, 128), W3r, Wm1, bm1.reshape(1, 128),
                    Wm2, bm2.reshape(1, 64), Wm3, bm3.reshape(1, 4))
    return out


# speedup vs baseline: 1.1563x; 1.1563x over previous
import functools

import jax
import jax.numpy as jnp
from jax import lax
from jax.experimental import pallas as pl
from jax.experimental.pallas import tpu as pltpu, tpu_sc as plsc

N = 10000
P = 320000
M = 512
NH = 10000

NC = 2
NS = 16
NW = NC * NS

RL = 128
RW = 80
PPW = RL * RW
PP = PPW * NW
NP = 10016
ROWS_PER_TILE = NP // NS
MROWS = 520
TRASH = 1024
ACC_ROWS = M + TRASH + 16

F32 = jnp.float32
I32 = jnp.int32

_mesh = plsc.VectorSubcoreMesh(core_axis_name="c", subcore_axis_name="s")


def _leaky(a):
    return jnp.where(a >= 0, a, 0.1 * a)


@functools.partial(
    pl.kernel,
    out_type=[
        jax.ShapeDtypeStruct((NP,), I32),
        jax.ShapeDtypeStruct((M,), I32),
        jax.ShapeDtypeStruct((M, 4), F32),
    ],
    mesh=_mesh,
    compiler_params=pltpu.CompilerParams(needs_layout_passes=False, use_tc_tiling_on_sc=False),
    scratch_types=[
        pltpu.VMEM((M,), I32),
        pltpu.VMEM((NP,), I32),
        pltpu.VMEM((M,), I32),
        pltpu.VMEM((M, 4), F32),
        pltpu.VMEM((NP * 2,), F32),
        pltpu.VMEM((NP * 2,), F32),
    ],
)
def _sc_prep(mi_hbm, x_hbm, fp_hbm, nm_hbm, ms_hbm, mg_hbm,
             mib, nmb, msb, mgb, xb, fpb):
    c = lax.axis_index("c")
    s = lax.axis_index("s")

    @pl.when(jnp.logical_and(c == 0, s == 0))
    def _():
        pltpu.sync_copy(mi_hbm, mib)
        pltpu.sync_copy(x_hbm, xb)
        pltpu.sync_copy(fp_hbm, fpb)
        mval = jnp.full((16,), M, I32)

        def fill(i, carry):
            nmb[pl.ds(i * 16, 16)] = mval
            return carry

        lax.fori_loop(0, NP // 16, fill, 0)
        lane = jnp.arange(16, dtype=I32)
        for i in range(M // 16):
            mi = mib[pl.ds(i * 16, 16)]
            plsc.store_scatter(nmb, [mi], i * 16 + lane)
        for i in range(M // 16):
            mi = mib[pl.ds(i * 16, 16)]
            slots = plsc.load_gather(nmb, [mi])
            msb[pl.ds(i * 16, 16)] = slots
            dst = i * 16 + lane
            for col in range(2):
                cc = jnp.full((16,), col, I32)
                plsc.store_scatter(mgb, [dst, cc], plsc.load_gather(fpb, [mi * 2 + col]))
                plsc.store_scatter(mgb, [dst, cc + 2], plsc.load_gather(xb, [mi * 2 + col]))
        pltpu.sync_copy(nmb, nm_hbm)
        pltpu.sync_copy(msb, ms_hbm)
        pltpu.sync_copy(mgb, mg_hbm)


def _tc_xin_body(x_ref, fp_ref, nm_ref, out_ref):
    flag = (nm_ref[...] != M).astype(F32)
    z = jnp.zeros((NP, 1), F32)
    o = jnp.ones((NP, 1), F32)
    out_ref[...] = jnp.concatenate(
        [x_ref[...], fp_ref[...], flag, z, z, z, z, o, z, z, z, z, z, z], axis=1)


_tc_xin = pl.pallas_call(
    _tc_xin_body, out_shape=jax.ShapeDtypeStruct((NP, 16), F32))


_mesh1 = plsc.VectorSubcoreMesh(
    core_axis_name="c", subcore_axis_name="s", num_cores=1)


def _make_stage_a(half):
    base_rows = half * (NW // 2) * RW

    @functools.partial(
        pl.kernel,
        out_type=jax.ShapeDtypeStruct((NP, 16), F32),
        mesh=_mesh1,
        compiler_params=pltpu.CompilerParams(needs_layout_passes=False, use_tc_tiling_on_sc=False),
        scratch_types=[
            pltpu.VMEM((RW, RL), I32),
            pltpu.VMEM((RW, RL), I32),
            pltpu.VMEM((RL, 16), F32),
            pltpu.VMEM((RL, 16), F32),
            pltpu.VMEM((RL, 16), F32),
            pltpu.VMEM((RL, 16), F32),
            pltpu.VMEM_SHARED((NP, 16), F32),
            pltpu.SemaphoreType.DMA,
            pltpu.SemaphoreType.DMA,
            pltpu.SemaphoreType.DMA,
            pltpu.SemaphoreType.DMA,
        ],
    )
    def _stage_a(xin_hbm, pfw_hbm, node_hbm, hedge_hbm, z_hbm, out_hbm,
                 nodeb, hedgeb, xr0, xr1, pf0, pf1, accS, sg0, sg1, sp0, sp1):
        s = lax.axis_index("s")
        rowbase = base_rows + s * RW
        pinbase = rowbase * RL

        pltpu.sync_copy(z_hbm.at[pl.ds(s * ROWS_PER_TILE, ROWS_PER_TILE)],
                        accS.at[pl.ds(s * ROWS_PER_TILE, ROWS_PER_TILE)])
        pltpu.sync_copy(node_hbm.at[pl.ds(rowbase, RW)], nodeb)
        pltpu.sync_copy(hedge_hbm.at[pl.ds(rowbase, RW)], hedgeb)
        plsc.subcore_barrier()

        def issue(j, xr, pf, sg, sp):
            pltpu.async_copy(xin_hbm.at[nodeb.at[j]], xr, sg)
            pltpu.async_copy(pfw_hbm.at[pl.ds(pinbase + j * RL, RL)], pf, sp)

        def drain(j, xr, pf, sg, sp):
            pltpu.make_async_copy(xin_hbm.at[nodeb.at[j]], xr, sg).wait()
            pltpu.make_async_copy(pfw_hbm.at[pl.ds(pinbase + j * RL, RL)], pf, sp).wait()
            pltpu.sync_copy(xr, accS.at[hedgeb.at[j]], add=True)
            pltpu.sync_copy(pf, accS.at[hedgeb.at[j]], add=True)

        issue(0, xr0, pf0, sg0, sp0)

        def body(jj, carry):
            j0 = jj * 2
            j1 = j0 + 1
            issue(j1, xr1, pf1, sg1, sp1)
            drain(j0, xr0, pf0, sg0, sp0)

            @pl.when(jj < RW // 2 - 1)
            def _():
                issue(j0 + 2, xr0, pf0, sg0, sp0)

            drain(j1, xr1, pf1, sg1, sp1)
            return carry

        lax.fori_loop(0, RW // 2, body, 0)
        plsc.subcore_barrier()
        pltpu.sync_copy(accS.at[pl.ds(s * ROWS_PER_TILE, ROWS_PER_TILE)],
                        out_hbm.at[pl.ds(s * ROWS_PER_TILE, ROWS_PER_TILE)])

    return _stage_a


_sc_stage_a0 = _make_stage_a(0)
_sc_stage_a1 = _make_stage_a(1)


def _tc_ubar_body(acc0_ref, acc1_ref, out_ref):
    a = acc0_ref[...] + acc1_ref[...]
    cnt = a[:, 9:10]
    inv = 1.0 / jnp.maximum(cnt, 1.0)
    t = jnp.where(cnt > 0.5, 1.0, 0.0)
    u = a[:, :9] * inv
    z = jnp.zeros((NP, 1), F32)
    out_ref[...] = jnp.concatenate([u, t, jnp.ones((NP, 1), F32), z, z, z, z, z], axis=1)


_tc_ubar = pl.pallas_call(
    _tc_ubar_body, out_shape=jax.ShapeDtypeStruct((NP, 16), F32))


def _make_stage_b(half):
    base_rows = half * (NW // 2) * RW

    @functools.partial(
        pl.kernel,
        out_type=jax.ShapeDtypeStruct((MROWS, 16), F32),
        mesh=_mesh1,
        compiler_params=pltpu.CompilerParams(needs_layout_passes=False, use_tc_tiling_on_sc=False),
        scratch_types=[
            pltpu.VMEM((NP,), I32),
            pltpu.VMEM((RW, RL), I32),
            pltpu.VMEM((RW, RL), I32),
            pltpu.VMEM((RL, 16), F32),
            pltpu.VMEM((RL, 16), F32),
            pltpu.VMEM((RL,), I32),
            pltpu.VMEM((RL,), I32),
            pltpu.VMEM_SHARED((ACC_ROWS, 16), F32),
            pltpu.SemaphoreType.DMA,
            pltpu.SemaphoreType.DMA,
        ],
    )
    def _stage_b(ubar_hbm, nm_hbm, node_hbm, hedge_hbm, z_hbm, out_hbm,
                 nmv, nodeb, hedgeb, ur0, ur1, sl0, sl1, accS, sg0, sg1):
        s = lax.axis_index("s")
        rowbase = base_rows + s * RW
        zrows = ACC_ROWS // NS

        pltpu.sync_copy(z_hbm.at[pl.ds(s * zrows, zrows)],
                        accS.at[pl.ds(s * zrows, zrows)])
        pltpu.sync_copy(nm_hbm, nmv)
        pltpu.sync_copy(node_hbm.at[pl.ds(rowbase, RW)], nodeb)
        pltpu.sync_copy(hedge_hbm.at[pl.ds(rowbase, RW)], hedgeb)
        plsc.subcore_barrier()

        def issue(j, ur, sg, slotb):
            pltpu.async_copy(ubar_hbm.at[hedgeb.at[j]], ur, sg)

            def inner(i, icarry):
                idx = nodeb[j, pl.ds(i * 16, 16)]
                sl = plsc.load_gather(nmv, [idx])
                slotb[pl.ds(i * 16, 16)] = jnp.where(
                    sl == M, M + jnp.bitwise_and(idx, TRASH - 1), sl)
                return icarry

            lax.fori_loop(0, RL // 16, inner, 0)

        def drain(j, ur, sg, slotb):
            pltpu.make_async_copy(ubar_hbm.at[hedgeb.at[j]], ur, sg).wait()
            pltpu.sync_copy(ur, accS.at[slotb], add=True)

        issue(0, ur0, sg0, sl0)

        def body(jj, carry):
            j0 = jj * 2
            j1 = j0 + 1
            issue(j1, ur1, sg1, sl1)
            drain(j0, ur0, sg0, sl0)

            @pl.when(jj < RW // 2 - 1)
            def _():
                issue(j0 + 2, ur0, sg0, sl0)

            drain(j1, ur1, sg1, sl1)
            return carry

        lax.fori_loop(0, RW // 2, body, 0)
        plsc.subcore_barrier()

        @pl.when(s < 13)
        def _():
            pltpu.sync_copy(accS.at[pl.ds(s * 40, 40)],
                            out_hbm.at[pl.ds(s * 40, 40)])

    return _stage_b


_sc_stage_b0 = _make_stage_b(0)
_sc_stage_b1 = _make_stage_b(1)


def _tc_macro_body(acc0_ref, acc1_ref, ms_ref, mg_ref,
                   W1_ref, b1_ref, Wp_ref, W2l_ref, b2l_ref, W2r_ref,
                   W3l_ref, b3l_ref, W3r_ref, Wm1_ref, bm1_ref,
                   Wm2_ref, bm2_ref, Wm3_ref, bm3_ref, out_ref):
    hi = lax.Precision.HIGHEST
    acc = acc0_ref[...] + acc1_ref[...]
    cols = lax.broadcasted_iota(I32, (M, MROWS), 1)
    onehot = (cols == ms_ref[...]).astype(F32)
    H = jnp.dot(onehot, acc, precision=hi)
    U5, U4 = H[:, 0:5], H[:, 5:9]
    T, C = H[:, 9:10], H[:, 10:11]
    pre = (jnp.dot(U5, W1_ref[...], precision=hi) + T * b1_ref[...]
           + jnp.dot(U4, Wp_ref[...], precision=hi))
    hm = _leaky(pre / jnp.maximum(C, 1.0))

    mg = mg_ref[...]
    px, py, sx, sy = mg[:, 0:1], mg[:, 1:2], mg[:, 2:3], mg[:, 3:4]
    ri = lax.broadcasted_iota(I32, (M, M), 0)
    ci = lax.broadcasted_iota(I32, (M, M), 1)
    eye = (ri == ci).astype(F32)
    noteye = 1.0 - eye
    dn = (((0,), (0,)), ((), ()))
    pxr = lax.dot_general(px, eye, dn, precision=hi)
    pyr = lax.dot_general(py, eye, dn, precision=hi)
    adj_v = ((px + sx) <= pxr).astype(F32) * noteye
    adj_h = ((py + sy) <= pyr).astype(F32) * noteye
    ones_col = jnp.ones((M, 1), F32)

    def sage(xf, adj, Wl, bl, Wr):
        deg = lax.dot_general(adj, ones_col, dn, precision=hi)
        agg = lax.dot_general(adj, xf, dn)
        return jnp.dot(agg / jnp.maximum(deg, 1.0), Wl) + bl + jnp.dot(xf, Wr)

    xv = _leaky(sage(hm, adj_v, W2l_ref[...], b2l_ref[...], W2r_ref[...]))
    xh = _leaky(sage(hm, adj_h, W2l_ref[...], b2l_ref[...], W2r_ref[...]))
    xc = jnp.concatenate([xv, xh], axis=1)
    xv2 = _leaky(sage(xc, adj_v, W3l_ref[...], b3l_ref[...], W3r_ref[...]))
    xh2 = _leaky(sage(xc, adj_h, W3l_ref[...], b3l_ref[...], W3r_ref[...]))
    xc2 = jnp.concatenate([xv2, xh2], axis=1)
    o = _leaky(jnp.dot(xc2, Wm1_ref[...]) + bm1_ref[...])
    o = _leaky(jnp.dot(o, Wm2_ref[...]) + bm2_ref[...])
    out_ref[...] = jnp.dot(o, Wm3_ref[...]) + bm3_ref[...]


_tc_macro = pl.pallas_call(
    _tc_macro_body, out_shape=jax.ShapeDtypeStruct((M, 4), F32))


def kernel(x, edge_index, pin_feature, batch, fake_pos, macro_index,
           W1, b1, Wp, W2l, b2l, W2r, W3l, b3l, W3r,
           Wm1, bm1, Wm2, bm2, Wm3, bm3):
    del batch
    node_idx = edge_index[0]
    hedge_idx = edge_index[1]

    pad = PP - P
    prng = jnp.arange(pad, dtype=I32)
    node_pa = jnp.concatenate([node_idx, prng % N]).reshape(NW * RW, RL)
    node_pb = jnp.concatenate([node_idx, N + (prng % 16)]).reshape(NW * RW, RL)
    hedge_p = jnp.concatenate([hedge_idx, NH + (prng % 16)]).reshape(NW * RW, RL)
    pfw = jnp.concatenate(
        [jnp.zeros((P, 5), F32), pin_feature, jnp.zeros((P, 7), F32)], axis=1)
    pfw = jnp.concatenate([pfw, jnp.zeros((pad, 16), F32)], axis=0)
    x_p = jnp.concatenate([x, jnp.zeros((NP - N, 2), F32)], axis=0)
    fp_p = jnp.concatenate([fake_pos, jnp.zeros((NP - N, 2), F32)], axis=0)
    zeros16 = jnp.zeros((NP, 16), F32)

    nodemap, macroslot, mgeo = _sc_prep(macro_index, x_p.reshape(-1), fp_p.reshape(-1))
    xin16 = _tc_xin(x_p, fp_p, nodemap.reshape(NP, 1))
    accA0 = _sc_stage_a0(xin16, pfw, node_pa, hedge_p, zeros16)
    accA1 = _sc_stage_a1(xin16, pfw, node_pa, hedge_p, zeros16)
    ubar = _tc_ubar(accA0, accA1)
    accB0 = _sc_stage_b0(ubar, nodemap, node_pb, hedge_p, zeros16)
    accB1 = _sc_stage_b1(ubar, nodemap, node_pb, hedge_p, zeros16)
    out = _tc_macro(accB0, accB1, macroslot.reshape(M, 1), mgeo,
                    W1, b1.reshape(1, 128), Wp, W2l, b2l.reshape(1, 128), W2r,
                    W3l, b3l.reshape(1, 128), W3r, Wm1, bm1.reshape(1, 128),
                    Wm2, bm2.reshape(1, 64), Wm3, bm3.reshape(1, 4))
    return out


# speedup vs baseline: 1.3873x; 1.1998x over previous
import functools

import jax
import jax.numpy as jnp
from jax import lax
from jax.experimental import pallas as pl
from jax.experimental.pallas import tpu as pltpu, tpu_sc as plsc

N = 10000
P = 320000
M = 512
NH = 10000

NC = 2
NS = 16
NW = NC * NS

RL = 128
RW = 80
PPW = RL * RW
PP = PPW * NW
NP = 10016
ROWS_PER_TILE = NP // NS
MROWS = 520
TRASH = 1024
ACC_ROWS = M + TRASH + 16

F32 = jnp.float32
I32 = jnp.int32

_mesh = plsc.VectorSubcoreMesh(core_axis_name="c", subcore_axis_name="s")


def _leaky(a):
    return jnp.where(a >= 0, a, 0.1 * a)


@functools.partial(
    pl.kernel,
    out_type=[
        jax.ShapeDtypeStruct((NP,), I32),
        jax.ShapeDtypeStruct((M,), I32),
        jax.ShapeDtypeStruct((M, 4), F32),
    ],
    mesh=_mesh,
    compiler_params=pltpu.CompilerParams(needs_layout_passes=False, use_tc_tiling_on_sc=False),
    scratch_types=[
        pltpu.VMEM((M,), I32),
        pltpu.VMEM((NP,), I32),
        pltpu.VMEM((M,), I32),
        pltpu.VMEM((M, 4), F32),
        pltpu.VMEM((NP * 2,), F32),
        pltpu.VMEM((NP * 2,), F32),
    ],
)
def _sc_prep(mi_hbm, x_hbm, fp_hbm, nm_hbm, ms_hbm, mg_hbm,
             mib, nmb, msb, mgb, xb, fpb):
    c = lax.axis_index("c")
    s = lax.axis_index("s")

    @pl.when(jnp.logical_and(c == 0, s == 0))
    def _():
        pltpu.sync_copy(mi_hbm, mib)
        pltpu.sync_copy(x_hbm, xb)
        pltpu.sync_copy(fp_hbm, fpb)
        mval = jnp.full((16,), M, I32)

        def fill(i, carry):
            nmb[pl.ds(i * 16, 16)] = mval
            return carry

        lax.fori_loop(0, NP // 16, fill, 0)
        lane = jnp.arange(16, dtype=I32)
        for i in range(M // 16):
            mi = mib[pl.ds(i * 16, 16)]
            plsc.store_scatter(nmb, [mi], i * 16 + lane)
        for i in range(M // 16):
            mi = mib[pl.ds(i * 16, 16)]
            slots = plsc.load_gather(nmb, [mi])
            msb[pl.ds(i * 16, 16)] = slots
            dst = i * 16 + lane
            for col in range(2):
                cc = jnp.full((16,), col, I32)
                plsc.store_scatter(mgb, [dst, cc], plsc.load_gather(fpb, [mi * 2 + col]))
                plsc.store_scatter(mgb, [dst, cc + 2], plsc.load_gather(xb, [mi * 2 + col]))
        pltpu.sync_copy(nmb, nm_hbm)
        pltpu.sync_copy(msb, ms_hbm)
        pltpu.sync_copy(mgb, mg_hbm)


def _tc_xin_body(x_ref, fp_ref, nm_ref, out_ref):
    flag = (nm_ref[...] != M).astype(F32)
    z = jnp.zeros((NP, 1), F32)
    o = jnp.ones((NP, 1), F32)
    out_ref[...] = jnp.concatenate(
        [x_ref[...], fp_ref[...], flag, z, z, z, z, o, z, z, z, z, z, z], axis=1)


_tc_xin = pl.pallas_call(
    _tc_xin_body, out_shape=jax.ShapeDtypeStruct((NP, 16), F32))


@functools.partial(
    pl.kernel,
    out_type=jax.ShapeDtypeStruct((NC, NP, 16), F32),
    mesh=_mesh,
    compiler_params=pltpu.CompilerParams(needs_layout_passes=False, use_tc_tiling_on_sc=False),
    scratch_types=[
        pltpu.VMEM((RW, RL), I32),
        pltpu.VMEM((RW, RL), I32),
        pltpu.VMEM((RL, 16), F32),
        pltpu.VMEM((RL, 16), F32),
        pltpu.VMEM((RL, 16), F32),
        pltpu.VMEM((RL, 16), F32),
        pltpu.VMEM_SHARED((NP, 16), F32),
        pltpu.SemaphoreType.DMA,
        pltpu.SemaphoreType.DMA,
        pltpu.SemaphoreType.DMA,
        pltpu.SemaphoreType.DMA,
    ],
)
def _sc_stage_a(xin_hbm, pfw_hbm, node_hbm, hedge_hbm, z_hbm, out_hbm,
                nodeb, hedgeb, xr0, xr1, pf0, pf1, accS, sg0, sg1, sp0, sp1):
    c = lax.axis_index("c")
    s = lax.axis_index("s")
    wid = s * NC + c
    rowbase = wid * RW
    pinbase = wid * PPW

    pltpu.sync_copy(z_hbm.at[pl.ds(s * ROWS_PER_TILE, ROWS_PER_TILE)],
                    accS.at[pl.ds(s * ROWS_PER_TILE, ROWS_PER_TILE)])
    pltpu.sync_copy(node_hbm.at[pl.ds(rowbase, RW)], nodeb)
    pltpu.sync_copy(hedge_hbm.at[pl.ds(rowbase, RW)], hedgeb)
    plsc.subcore_barrier()

    def issue(j, xr, pf, sg, sp):
        pltpu.async_copy(xin_hbm.at[nodeb.at[j]], xr, sg)
        pltpu.async_copy(pfw_hbm.at[pl.ds(pinbase + j * RL, RL)], pf, sp)

    def drain(j, xr, pf, sg, sp):
        pltpu.make_async_copy(xin_hbm.at[nodeb.at[j]], xr, sg).wait()
        pltpu.make_async_copy(pfw_hbm.at[pl.ds(pinbase + j * RL, RL)], pf, sp).wait()
        pltpu.sync_copy(xr, accS.at[hedgeb.at[j]], add=True)
        pltpu.sync_copy(pf, accS.at[hedgeb.at[j]], add=True)

    issue(0, xr0, pf0, sg0, sp0)

    def body(jj, carry):
        j0 = jj * 2
        j1 = j0 + 1
        issue(j1, xr1, pf1, sg1, sp1)
        drain(j0, xr0, pf0, sg0, sp0)

        @pl.when(jj < RW // 2 - 1)
        def _():
            issue(j0 + 2, xr0, pf0, sg0, sp0)

        drain(j1, xr1, pf1, sg1, sp1)
        return carry

    lax.fori_loop(0, RW // 2, body, 0)
    plsc.subcore_barrier()
    pltpu.sync_copy(accS.at[pl.ds(s * ROWS_PER_TILE, ROWS_PER_TILE)],
                    out_hbm.at[c, pl.ds(s * ROWS_PER_TILE, ROWS_PER_TILE)])


def _tc_ubar_body(acc_ref, out_ref):
    a = acc_ref[0] + acc_ref[1]
    cnt = a[:, 9:10]
    inv = 1.0 / jnp.maximum(cnt, 1.0)
    t = jnp.where(cnt > 0.5, 1.0, 0.0)
    u = a[:, :9] * inv
    z = jnp.zeros((NP, 1), F32)
    out_ref[...] = jnp.concatenate([u, t, jnp.ones((NP, 1), F32), z, z, z, z, z], axis=1)


_tc_ubar = pl.pallas_call(
    _tc_ubar_body, out_shape=jax.ShapeDtypeStruct((NP, 16), F32))


@functools.partial(
    pl.kernel,
    out_type=jax.ShapeDtypeStruct((NC, MROWS, 16), F32),
    mesh=_mesh,
    compiler_params=pltpu.CompilerParams(needs_layout_passes=False, use_tc_tiling_on_sc=False),
    scratch_types=[
        pltpu.VMEM((NP,), I32),
        pltpu.VMEM((RW, RL), I32),
        pltpu.VMEM((RW, RL), I32),
        pltpu.VMEM((RL, 16), F32),
        pltpu.VMEM((RL, 16), F32),
        pltpu.VMEM((RL,), I32),
        pltpu.VMEM((RL,), I32),
        pltpu.VMEM_SHARED((ACC_ROWS, 16), F32),
        pltpu.SemaphoreType.DMA,
        pltpu.SemaphoreType.DMA,
    ],
)
def _sc_stage_b(ubar_hbm, nm_hbm, node_hbm, hedge_hbm, z_hbm, out_hbm,
                nmv, nodeb, hedgeb, ur0, ur1, sl0, sl1, accS, sg0, sg1):
    c = lax.axis_index("c")
    s = lax.axis_index("s")
    wid = s * NC + c
    rowbase = wid * RW
    zrows = ACC_ROWS // NS

    pltpu.sync_copy(z_hbm.at[pl.ds(s * zrows, zrows)],
                    accS.at[pl.ds(s * zrows, zrows)])
    pltpu.sync_copy(nm_hbm, nmv)
    pltpu.sync_copy(node_hbm.at[pl.ds(rowbase, RW)], nodeb)
    pltpu.sync_copy(hedge_hbm.at[pl.ds(rowbase, RW)], hedgeb)
    plsc.subcore_barrier()

    def issue(j, ur, sg, slotb):
        pltpu.async_copy(ubar_hbm.at[hedgeb.at[j]], ur, sg)

        def inner(i, icarry):
            idx = nodeb[j, pl.ds(i * 16, 16)]
            sl = plsc.load_gather(nmv, [idx])
            slotb[pl.ds(i * 16, 16)] = jnp.where(
                sl == M, M + jnp.bitwise_and(idx, TRASH - 1), sl)
            return icarry

        lax.fori_loop(0, RL // 16, inner, 0)

    def drain(j, ur, sg, slotb):
        pltpu.make_async_copy(ubar_hbm.at[hedgeb.at[j]], ur, sg).wait()
        pltpu.sync_copy(ur, accS.at[slotb], add=True)

    issue(0, ur0, sg0, sl0)

    def body(jj, carry):
        j0 = jj * 2
        j1 = j0 + 1
        issue(j1, ur1, sg1, sl1)
        drain(j0, ur0, sg0, sl0)

        @pl.when(jj < RW // 2 - 1)
        def _():
            issue(j0 + 2, ur0, sg0, sl0)

        drain(j1, ur1, sg1, sl1)
        return carry

    lax.fori_loop(0, RW // 2, body, 0)
    plsc.subcore_barrier()

    @pl.when(s < 13)
    def _():
        pltpu.sync_copy(accS.at[pl.ds(s * 40, 40)],
                        out_hbm.at[c, pl.ds(s * 40, 40)])


def _tc_macro_body(acc_ref, ms_ref, mg_ref,
                   W1_ref, b1_ref, Wp_ref, W2l_ref, b2l_ref, W2r_ref,
                   W3l_ref, b3l_ref, W3r_ref, Wm1_ref, bm1_ref,
                   Wm2_ref, bm2_ref, Wm3_ref, bm3_ref, out_ref):
    hi = lax.Precision.HIGHEST
    acc = acc_ref[0] + acc_ref[1]
    cols = lax.broadcasted_iota(I32, (M, MROWS), 1)
    onehot = (cols == ms_ref[...]).astype(F32)
    H = jnp.dot(onehot, acc, precision=hi)
    U5, U4 = H[:, 0:5], H[:, 5:9]
    T, C = H[:, 9:10], H[:, 10:11]
    pre = (jnp.dot(U5, W1_ref[...], precision=hi) + T * b1_ref[...]
           + jnp.dot(U4, Wp_ref[...], precision=hi))
    hm = _leaky(pre / jnp.maximum(C, 1.0))

    mg = mg_ref[...]
    px, py, sx, sy = mg[:, 0:1], mg[:, 1:2], mg[:, 2:3], mg[:, 3:4]
    ri = lax.broadcasted_iota(I32, (M, M), 0)
    ci = lax.broadcasted_iota(I32, (M, M), 1)
    eye = (ri == ci).astype(F32)
    noteye = 1.0 - eye
    dn = (((0,), (0,)), ((), ()))
    pxr = lax.dot_general(px, eye, dn, precision=hi)
    pyr = lax.dot_general(py, eye, dn, precision=hi)
    adj_v = ((px + sx) <= pxr).astype(F32) * noteye
    adj_h = ((py + sy) <= pyr).astype(F32) * noteye
    ones_col = jnp.ones((M, 1), F32)

    def sage(xf, adj, Wl, bl, Wr):
        deg = lax.dot_general(adj, ones_col, dn, precision=hi)
        agg = lax.dot_general(adj, xf, dn)
        return jnp.dot(agg / jnp.maximum(deg, 1.0), Wl) + bl + jnp.dot(xf, Wr)

    xv = _leaky(sage(hm, adj_v, W2l_ref[...], b2l_ref[...], W2r_ref[...]))
    xh = _leaky(sage(hm, adj_h, W2l_ref[...], b2l_ref[...], W2r_ref[...]))
    xc = jnp.concatenate([xv, xh], axis=1)
    xv2 = _leaky(sage(xc, adj_v, W3l_ref[...], b3l_ref[...], W3r_ref[...]))
    xh2 = _leaky(sage(xc, adj_h, W3l_ref[...], b3l_ref[...], W3r_ref[...]))
    xc2 = jnp.concatenate([xv2, xh2], axis=1)
    o = _leaky(jnp.dot(xc2, Wm1_ref[...]) + bm1_ref[...])
    o = _leaky(jnp.dot(o, Wm2_ref[...]) + bm2_ref[...])
    out_ref[...] = jnp.dot(o, Wm3_ref[...]) + bm3_ref[...]


_tc_macro = pl.pallas_call(
    _tc_macro_body, out_shape=jax.ShapeDtypeStruct((M, 4), F32))


def kernel(x, edge_index, pin_feature, batch, fake_pos, macro_index,
           W1, b1, Wp, W2l, b2l, W2r, W3l, b3l, W3r,
           Wm1, bm1, Wm2, bm2, Wm3, bm3):
    del batch
    node_idx = edge_index[0]
    hedge_idx = edge_index[1]

    pad = PP - P
    prng = jnp.arange(pad, dtype=I32)
    node_pa = jnp.concatenate([node_idx, prng % N]).reshape(NW * RW, RL)
    node_pb = jnp.concatenate([node_idx, N + (prng % 16)]).reshape(NW * RW, RL)
    hedge_p = jnp.concatenate([hedge_idx, NH + (prng % 16)]).reshape(NW * RW, RL)
    pfw = jnp.concatenate(
        [jnp.zeros((P, 5), F32), pin_feature, jnp.zeros((P, 7), F32)], axis=1)
    pfw = jnp.concatenate([pfw, jnp.zeros((pad, 16), F32)], axis=0)
    x_p = jnp.concatenate([x, jnp.zeros((NP - N, 2), F32)], axis=0)
    fp_p = jnp.concatenate([fake_pos, jnp.zeros((NP - N, 2), F32)], axis=0)
    zeros16 = jnp.zeros((NP, 16), F32)

    nodemap, macroslot, mgeo = _sc_prep(macro_index, x_p.reshape(-1), fp_p.reshape(-1))
    xin16 = _tc_xin(x_p, fp_p, nodemap.reshape(NP, 1))
    accA = _sc_stage_a(xin16, pfw, node_pa, hedge_p, zeros16)
    ubar = _tc_ubar(accA)
    accB = _sc_stage_b(ubar, nodemap, node_pb, hedge_p, zeros16)
    out = _tc_macro(accB, macroslot.reshape(M, 1), mgeo,
                    W1, b1.reshape(1, 128), Wp, W2l, b2l.reshape(1, 128), W2r,
                    W3l, b3l.reshape(1, 128), W3r, Wm1, bm1.reshape(1, 128),
                    Wm2, bm2.reshape(1, 64), Wm3, bm3.reshape(1, 4))
    return out


# speedup vs baseline: 1.4512x; 1.0461x over previous
import functools

import jax
import jax.numpy as jnp
from jax import lax
from jax.experimental import pallas as pl
from jax.experimental.pallas import tpu as pltpu, tpu_sc as plsc

N = 10000
P = 320000
M = 512
NH = 10000

NC = 2
NS = 16
NW = NC * NS

RL = 256
RW = 40
PPW = RL * RW
PP = PPW * NW
NP = 10016
ROWS_PER_TILE = NP // NS
MROWS = 520
TRASH = 1024
ACC_ROWS = M + TRASH + 16

F32 = jnp.float32
I32 = jnp.int32

_mesh = plsc.VectorSubcoreMesh(core_axis_name="c", subcore_axis_name="s")


def _leaky(a):
    return jnp.where(a >= 0, a, 0.1 * a)


@functools.partial(
    pl.kernel,
    out_type=[
        jax.ShapeDtypeStruct((NP,), I32),
        jax.ShapeDtypeStruct((M,), I32),
        jax.ShapeDtypeStruct((M, 4), F32),
    ],
    mesh=_mesh,
    compiler_params=pltpu.CompilerParams(needs_layout_passes=False, use_tc_tiling_on_sc=False),
    scratch_types=[
        pltpu.VMEM((M,), I32),
        pltpu.VMEM((NP,), I32),
        pltpu.VMEM((M,), I32),
        pltpu.VMEM((M, 4), F32),
        pltpu.VMEM((NP * 2,), F32),
        pltpu.VMEM((NP * 2,), F32),
    ],
)
def _sc_prep(mi_hbm, x_hbm, fp_hbm, nm_hbm, ms_hbm, mg_hbm,
             mib, nmb, msb, mgb, xb, fpb):
    c = lax.axis_index("c")
    s = lax.axis_index("s")

    @pl.when(jnp.logical_and(c == 0, s == 0))
    def _():
        pltpu.sync_copy(mi_hbm, mib)
        pltpu.sync_copy(x_hbm, xb)
        pltpu.sync_copy(fp_hbm, fpb)
        mval = jnp.full((16,), M, I32)

        def fill(i, carry):
            nmb[pl.ds(i * 16, 16)] = mval
            return carry

        lax.fori_loop(0, NP // 16, fill, 0)
        lane = jnp.arange(16, dtype=I32)
        for i in range(M // 16):
            mi = mib[pl.ds(i * 16, 16)]
            plsc.store_scatter(nmb, [mi], i * 16 + lane)
        for i in range(M // 16):
            mi = mib[pl.ds(i * 16, 16)]
            slots = plsc.load_gather(nmb, [mi])
            msb[pl.ds(i * 16, 16)] = slots
            dst = i * 16 + lane
            for col in range(2):
                cc = jnp.full((16,), col, I32)
                plsc.store_scatter(mgb, [dst, cc], plsc.load_gather(fpb, [mi * 2 + col]))
                plsc.store_scatter(mgb, [dst, cc + 2], plsc.load_gather(xb, [mi * 2 + col]))
        pltpu.sync_copy(nmb, nm_hbm)
        pltpu.sync_copy(msb, ms_hbm)
        pltpu.sync_copy(mgb, mg_hbm)


def _tc_xin_body(x_ref, fp_ref, nm_ref, out_ref):
    flag = (nm_ref[...] != M).astype(F32)
    z = jnp.zeros((NP, 1), F32)
    o = jnp.ones((NP, 1), F32)
    out_ref[...] = jnp.concatenate(
        [x_ref[...], fp_ref[...], flag, z, z, z, z, o, z, z, z, z, z, z], axis=1)


_tc_xin = pl.pallas_call(
    _tc_xin_body, out_shape=jax.ShapeDtypeStruct((NP, 16), F32))


@functools.partial(
    pl.kernel,
    out_type=jax.ShapeDtypeStruct((NC, NP, 16), F32),
    mesh=_mesh,
    compiler_params=pltpu.CompilerParams(needs_layout_passes=False, use_tc_tiling_on_sc=False),
    scratch_types=[
        pltpu.VMEM((RW, RL), I32),
        pltpu.VMEM((RW, RL), I32),
        pltpu.VMEM((RL, 16), F32),
        pltpu.VMEM((RL, 16), F32),
        pltpu.VMEM((RL, 16), F32),
        pltpu.VMEM((RL, 16), F32),
        pltpu.VMEM_SHARED((NP, 16), F32),
        pltpu.SemaphoreType.DMA,
        pltpu.SemaphoreType.DMA,
        pltpu.SemaphoreType.DMA,
        pltpu.SemaphoreType.DMA,
    ],
)
def _sc_stage_a(xin_hbm, pfw_hbm, node_hbm, hedge_hbm, z_hbm, out_hbm,
                nodeb, hedgeb, xr0, xr1, pf0, pf1, accS, sg0, sg1, sp0, sp1):
    c = lax.axis_index("c")
    s = lax.axis_index("s")
    wid = s * NC + c
    rowbase = wid * RW
    pinbase = wid * PPW

    pltpu.sync_copy(z_hbm.at[pl.ds(s * ROWS_PER_TILE, ROWS_PER_TILE)],
                    accS.at[pl.ds(s * ROWS_PER_TILE, ROWS_PER_TILE)])
    pltpu.sync_copy(node_hbm.at[pl.ds(rowbase, RW)], nodeb)
    pltpu.sync_copy(hedge_hbm.at[pl.ds(rowbase, RW)], hedgeb)
    plsc.subcore_barrier()

    def issue(j, xr, pf, sg, sp):
        pltpu.async_copy(xin_hbm.at[nodeb.at[j]], xr, sg)
        pltpu.async_copy(pfw_hbm.at[pl.ds(pinbase + j * RL, RL)], pf, sp)

    def drain(j, xr, pf, sg, sp):
        pltpu.make_async_copy(xin_hbm.at[nodeb.at[j]], xr, sg).wait()
        pltpu.make_async_copy(pfw_hbm.at[pl.ds(pinbase + j * RL, RL)], pf, sp).wait()
        pltpu.sync_copy(xr, accS.at[hedgeb.at[j]], add=True)
        pltpu.sync_copy(pf, accS.at[hedgeb.at[j]], add=True)

    issue(0, xr0, pf0, sg0, sp0)

    def body(jj, carry):
        j0 = jj * 2
        j1 = j0 + 1
        issue(j1, xr1, pf1, sg1, sp1)
        drain(j0, xr0, pf0, sg0, sp0)

        @pl.when(jj < RW // 2 - 1)
        def _():
            issue(j0 + 2, xr0, pf0, sg0, sp0)

        drain(j1, xr1, pf1, sg1, sp1)
        return carry

    lax.fori_loop(0, RW // 2, body, 0)
    plsc.subcore_barrier()
    pltpu.sync_copy(accS.at[pl.ds(s * ROWS_PER_TILE, ROWS_PER_TILE)],
                    out_hbm.at[c, pl.ds(s * ROWS_PER_TILE, ROWS_PER_TILE)])


def _tc_ubar_body(acc_ref, out_ref):
    a = acc_ref[0] + acc_ref[1]
    cnt = a[:, 9:10]
    inv = 1.0 / jnp.maximum(cnt, 1.0)
    t = jnp.where(cnt > 0.5, 1.0, 0.0)
    u = a[:, :9] * inv
    z = jnp.zeros((NP, 1), F32)
    out_ref[...] = jnp.concatenate([u, t, jnp.ones((NP, 1), F32), z, z, z, z, z], axis=1)


_tc_ubar = pl.pallas_call(
    _tc_ubar_body, out_shape=jax.ShapeDtypeStruct((NP, 16), F32))


@functools.partial(
    pl.kernel,
    out_type=jax.ShapeDtypeStruct((NC, MROWS, 16), F32),
    mesh=_mesh,
    compiler_params=pltpu.CompilerParams(needs_layout_passes=False, use_tc_tiling_on_sc=False),
    scratch_types=[
        pltpu.VMEM((NP,), I32),
        pltpu.VMEM((RW, RL), I32),
        pltpu.VMEM((RW, RL), I32),
        pltpu.VMEM((RL, 16), F32),
        pltpu.VMEM((RL, 16), F32),
        pltpu.VMEM((RL,), I32),
        pltpu.VMEM((RL,), I32),
        pltpu.VMEM_SHARED((ACC_ROWS, 16), F32),
        pltpu.SemaphoreType.DMA,
        pltpu.SemaphoreType.DMA,
    ],
)
def _sc_stage_b(ubar_hbm, nm_hbm, node_hbm, hedge_hbm, z_hbm, out_hbm,
                nmv, nodeb, hedgeb, ur0, ur1, sl0, sl1, accS, sg0, sg1):
    c = lax.axis_index("c")
    s = lax.axis_index("s")
    wid = s * NC + c
    rowbase = wid * RW
    zrows = ACC_ROWS // NS

    pltpu.sync_copy(z_hbm.at[pl.ds(s * zrows, zrows)],
                    accS.at[pl.ds(s * zrows, zrows)])
    pltpu.sync_copy(nm_hbm, nmv)
    pltpu.sync_copy(node_hbm.at[pl.ds(rowbase, RW)], nodeb)
    pltpu.sync_copy(hedge_hbm.at[pl.ds(rowbase, RW)], hedgeb)
    plsc.subcore_barrier()

    def issue(j, ur, sg, slotb):
        pltpu.async_copy(ubar_hbm.at[hedgeb.at[j]], ur, sg)

        def inner(i, icarry):
            idx = nodeb[j, pl.ds(i * 16, 16)]
            sl = plsc.load_gather(nmv, [idx])
            slotb[pl.ds(i * 16, 16)] = jnp.where(
                sl == M, M + jnp.bitwise_and(idx, TRASH - 1), sl)
            return icarry

        lax.fori_loop(0, RL // 16, inner, 0)

    def drain(j, ur, sg, slotb):
        pltpu.make_async_copy(ubar_hbm.at[hedgeb.at[j]], ur, sg).wait()
        pltpu.sync_copy(ur, accS.at[slotb], add=True)

    issue(0, ur0, sg0, sl0)

    def body(jj, carry):
        j0 = jj * 2
        j1 = j0 + 1
        issue(j1, ur1, sg1, sl1)
        drain(j0, ur0, sg0, sl0)

        @pl.when(jj < RW // 2 - 1)
        def _():
            issue(j0 + 2, ur0, sg0, sl0)

        drain(j1, ur1, sg1, sl1)
        return carry

    lax.fori_loop(0, RW // 2, body, 0)
    plsc.subcore_barrier()

    @pl.when(s < 13)
    def _():
        pltpu.sync_copy(accS.at[pl.ds(s * 40, 40)],
                        out_hbm.at[c, pl.ds(s * 40, 40)])


def _tc_macro_body(acc_ref, ms_ref, mg_ref,
                   W1_ref, b1_ref, Wp_ref, W2l_ref, b2l_ref, W2r_ref,
                   W3l_ref, b3l_ref, W3r_ref, Wm1_ref, bm1_ref,
                   Wm2_ref, bm2_ref, Wm3_ref, bm3_ref, out_ref):
    hi = lax.Precision.HIGHEST
    acc = acc_ref[0] + acc_ref[1]
    cols = lax.broadcasted_iota(I32, (M, MROWS), 1)
    onehot = (cols == ms_ref[...]).astype(F32)
    H = jnp.dot(onehot, acc, precision=hi)
    U5, U4 = H[:, 0:5], H[:, 5:9]
    T, C = H[:, 9:10], H[:, 10:11]
    pre = (jnp.dot(U5, W1_ref[...], precision=hi) + T * b1_ref[...]
           + jnp.dot(U4, Wp_ref[...], precision=hi))
    hm = _leaky(pre / jnp.maximum(C, 1.0))

    mg = mg_ref[...]
    px, py, sx, sy = mg[:, 0:1], mg[:, 1:2], mg[:, 2:3], mg[:, 3:4]
    ri = lax.broadcasted_iota(I32, (M, M), 0)
    ci = lax.broadcasted_iota(I32, (M, M), 1)
    eye = (ri == ci).astype(F32)
    noteye = 1.0 - eye
    dn = (((0,), (0,)), ((), ()))
    pxr = lax.dot_general(px, eye, dn, precision=hi)
    pyr = lax.dot_general(py, eye, dn, precision=hi)
    adj_v = ((px + sx) <= pxr).astype(F32) * noteye
    adj_h = ((py + sy) <= pyr).astype(F32) * noteye
    ones_col = jnp.ones((M, 1), F32)

    def sage(xf, adj, Wl, bl, Wr):
        deg = lax.dot_general(adj, ones_col, dn, precision=hi)
        agg = lax.dot_general(adj, xf, dn)
        return jnp.dot(agg / jnp.maximum(deg, 1.0), Wl) + bl + jnp.dot(xf, Wr)

    xv = _leaky(sage(hm, adj_v, W2l_ref[...], b2l_ref[...], W2r_ref[...]))
    xh = _leaky(sage(hm, adj_h, W2l_ref[...], b2l_ref[...], W2r_ref[...]))
    xc = jnp.concatenate([xv, xh], axis=1)
    xv2 = _leaky(sage(xc, adj_v, W3l_ref[...], b3l_ref[...], W3r_ref[...]))
    xh2 = _leaky(sage(xc, adj_h, W3l_ref[...], b3l_ref[...], W3r_ref[...]))
    xc2 = jnp.concatenate([xv2, xh2], axis=1)
    o = _leaky(jnp.dot(xc2, Wm1_ref[...]) + bm1_ref[...])
    o = _leaky(jnp.dot(o, Wm2_ref[...]) + bm2_ref[...])
    out_ref[...] = jnp.dot(o, Wm3_ref[...]) + bm3_ref[...]


_tc_macro = pl.pallas_call(
    _tc_macro_body, out_shape=jax.ShapeDtypeStruct((M, 4), F32))


def kernel(x, edge_index, pin_feature, batch, fake_pos, macro_index,
           W1, b1, Wp, W2l, b2l, W2r, W3l, b3l, W3r,
           Wm1, bm1, Wm2, bm2, Wm3, bm3):
    del batch
    node_idx = edge_index[0]
    hedge_idx = edge_index[1]

    pad = PP - P
    prng = jnp.arange(pad, dtype=I32)
    node_pa = jnp.concatenate([node_idx, prng % N]).reshape(NW * RW, RL)
    node_pb = jnp.concatenate([node_idx, N + (prng % 16)]).reshape(NW * RW, RL)
    hedge_p = jnp.concatenate([hedge_idx, NH + (prng % 16)]).reshape(NW * RW, RL)
    pfw = jnp.concatenate(
        [jnp.zeros((P, 5), F32), pin_feature, jnp.zeros((P, 7), F32)], axis=1)
    pfw = jnp.concatenate([pfw, jnp.zeros((pad, 16), F32)], axis=0)
    x_p = jnp.concatenate([x, jnp.zeros((NP - N, 2), F32)], axis=0)
    fp_p = jnp.concatenate([fake_pos, jnp.zeros((NP - N, 2), F32)], axis=0)
    zeros16 = jnp.zeros((NP, 16), F32)

    nodemap, macroslot, mgeo = _sc_prep(macro_index, x_p.reshape(-1), fp_p.reshape(-1))
    xin16 = _tc_xin(x_p, fp_p, nodemap.reshape(NP, 1))
    accA = _sc_stage_a(xin16, pfw, node_pa, hedge_p, zeros16)
    ubar = _tc_ubar(accA)
    accB = _sc_stage_b(ubar, nodemap, node_pb, hedge_p, zeros16)
    out = _tc_macro(accB, macroslot.reshape(M, 1), mgeo,
                    W1, b1.reshape(1, 128), Wp, W2l, b2l.reshape(1, 128), W2r,
                    W3l, b3l.reshape(1, 128), W3r, Wm1, bm1.reshape(1, 128),
                    Wm2, bm2.reshape(1, 64), Wm3, bm3.reshape(1, 4))
    return out


# speedup vs baseline: 1.4700x; 1.0129x over previous
import functools

import jax
import jax.numpy as jnp
from jax import lax
from jax.experimental import pallas as pl
from jax.experimental.pallas import tpu as pltpu, tpu_sc as plsc

N = 10000
P = 320000
M = 512
NH = 10000

NC = 2
NS = 16
NW = NC * NS

RL = 512
RW = 20
PPW = RL * RW
PP = PPW * NW
NP = 10016
ROWS_PER_TILE = NP // NS
MROWS = 520
TRASH = 1024
ACC_ROWS = M + TRASH + 16

F32 = jnp.float32
I32 = jnp.int32

_mesh = plsc.VectorSubcoreMesh(core_axis_name="c", subcore_axis_name="s")


def _leaky(a):
    return jnp.where(a >= 0, a, 0.1 * a)


@functools.partial(
    pl.kernel,
    out_type=[
        jax.ShapeDtypeStruct((NP,), I32),
        jax.ShapeDtypeStruct((M,), I32),
        jax.ShapeDtypeStruct((M, 4), F32),
    ],
    mesh=_mesh,
    compiler_params=pltpu.CompilerParams(needs_layout_passes=False, use_tc_tiling_on_sc=False),
    scratch_types=[
        pltpu.VMEM((M,), I32),
        pltpu.VMEM((NP,), I32),
        pltpu.VMEM((M,), I32),
        pltpu.VMEM((M, 4), F32),
        pltpu.VMEM((NP * 2,), F32),
        pltpu.VMEM((NP * 2,), F32),
    ],
)
def _sc_prep(mi_hbm, x_hbm, fp_hbm, nm_hbm, ms_hbm, mg_hbm,
             mib, nmb, msb, mgb, xb, fpb):
    c = lax.axis_index("c")
    s = lax.axis_index("s")

    @pl.when(jnp.logical_and(c == 0, s == 0))
    def _():
        pltpu.sync_copy(mi_hbm, mib)
        pltpu.sync_copy(x_hbm, xb)
        pltpu.sync_copy(fp_hbm, fpb)
        mval = jnp.full((16,), M, I32)

        def fill(i, carry):
            nmb[pl.ds(i * 16, 16)] = mval
            return carry

        lax.fori_loop(0, NP // 16, fill, 0)
        lane = jnp.arange(16, dtype=I32)
        for i in range(M // 16):
            mi = mib[pl.ds(i * 16, 16)]
            plsc.store_scatter(nmb, [mi], i * 16 + lane)
        for i in range(M // 16):
            mi = mib[pl.ds(i * 16, 16)]
            slots = plsc.load_gather(nmb, [mi])
            msb[pl.ds(i * 16, 16)] = slots
            dst = i * 16 + lane
            for col in range(2):
                cc = jnp.full((16,), col, I32)
                plsc.store_scatter(mgb, [dst, cc], plsc.load_gather(fpb, [mi * 2 + col]))
                plsc.store_scatter(mgb, [dst, cc + 2], plsc.load_gather(xb, [mi * 2 + col]))
        pltpu.sync_copy(nmb, nm_hbm)
        pltpu.sync_copy(msb, ms_hbm)
        pltpu.sync_copy(mgb, mg_hbm)


def _tc_xin_body(x_ref, fp_ref, nm_ref, out_ref):
    flag = (nm_ref[...] != M).astype(F32)
    z = jnp.zeros((NP, 1), F32)
    o = jnp.ones((NP, 1), F32)
    out_ref[...] = jnp.concatenate(
        [x_ref[...], fp_ref[...], flag, z, z, z, z, o, z, z, z, z, z, z], axis=1)


_tc_xin = pl.pallas_call(
    _tc_xin_body, out_shape=jax.ShapeDtypeStruct((NP, 16), F32))


@functools.partial(
    pl.kernel,
    out_type=jax.ShapeDtypeStruct((NC, NP, 16), F32),
    mesh=_mesh,
    compiler_params=pltpu.CompilerParams(needs_layout_passes=False, use_tc_tiling_on_sc=False),
    scratch_types=[
        pltpu.VMEM((RW, RL), I32),
        pltpu.VMEM((RW, RL), I32),
        pltpu.VMEM((RL, 16), F32),
        pltpu.VMEM((RL, 16), F32),
        pltpu.VMEM((RL, 16), F32),
        pltpu.VMEM((RL, 16), F32),
        pltpu.VMEM_SHARED((NP, 16), F32),
        pltpu.SemaphoreType.DMA,
        pltpu.SemaphoreType.DMA,
        pltpu.SemaphoreType.DMA,
        pltpu.SemaphoreType.DMA,
    ],
)
def _sc_stage_a(xin_hbm, pfw_hbm, node_hbm, hedge_hbm, z_hbm, out_hbm,
                nodeb, hedgeb, xr0, xr1, pf0, pf1, accS, sg0, sg1, sp0, sp1):
    c = lax.axis_index("c")
    s = lax.axis_index("s")
    wid = s * NC + c
    rowbase = wid * RW
    pinbase = wid * PPW

    pltpu.sync_copy(z_hbm.at[pl.ds(s * ROWS_PER_TILE, ROWS_PER_TILE)],
                    accS.at[pl.ds(s * ROWS_PER_TILE, ROWS_PER_TILE)])
    pltpu.sync_copy(node_hbm.at[pl.ds(rowbase, RW)], nodeb)
    pltpu.sync_copy(hedge_hbm.at[pl.ds(rowbase, RW)], hedgeb)
    plsc.subcore_barrier()

    def issue(j, xr, pf, sg, sp):
        pltpu.async_copy(xin_hbm.at[nodeb.at[j]], xr, sg)
        pltpu.async_copy(pfw_hbm.at[pl.ds(pinbase + j * RL, RL)], pf, sp)

    def drain(j, xr, pf, sg, sp):
        pltpu.make_async_copy(xin_hbm.at[nodeb.at[j]], xr, sg).wait()
        pltpu.make_async_copy(pfw_hbm.at[pl.ds(pinbase + j * RL, RL)], pf, sp).wait()
        pltpu.sync_copy(xr, accS.at[hedgeb.at[j]], add=True)
        pltpu.sync_copy(pf, accS.at[hedgeb.at[j]], add=True)

    issue(0, xr0, pf0, sg0, sp0)

    def body(jj, carry):
        j0 = jj * 2
        j1 = j0 + 1
        issue(j1, xr1, pf1, sg1, sp1)
        drain(j0, xr0, pf0, sg0, sp0)

        @pl.when(jj < RW // 2 - 1)
        def _():
            issue(j0 + 2, xr0, pf0, sg0, sp0)

        drain(j1, xr1, pf1, sg1, sp1)
        return carry

    lax.fori_loop(0, RW // 2, body, 0)
    plsc.subcore_barrier()
    pltpu.sync_copy(accS.at[pl.ds(s * ROWS_PER_TILE, ROWS_PER_TILE)],
                    out_hbm.at[c, pl.ds(s * ROWS_PER_TILE, ROWS_PER_TILE)])


def _tc_ubar_body(acc_ref, out_ref):
    a = acc_ref[0] + acc_ref[1]
    cnt = a[:, 9:10]
    inv = 1.0 / jnp.maximum(cnt, 1.0)
    t = jnp.where(cnt > 0.5, 1.0, 0.0)
    u = a[:, :9] * inv
    z = jnp.zeros((NP, 1), F32)
    out_ref[...] = jnp.concatenate([u, t, jnp.ones((NP, 1), F32), z, z, z, z, z], axis=1)


_tc_ubar = pl.pallas_call(
    _tc_ubar_body, out_shape=jax.ShapeDtypeStruct((NP, 16), F32))


@functools.partial(
    pl.kernel,
    out_type=jax.ShapeDtypeStruct((NC, MROWS, 16), F32),
    mesh=_mesh,
    compiler_params=pltpu.CompilerParams(needs_layout_passes=False, use_tc_tiling_on_sc=False),
    scratch_types=[
        pltpu.VMEM((NP,), I32),
        pltpu.VMEM((RW, RL), I32),
        pltpu.VMEM((RW, RL), I32),
        pltpu.VMEM((RL, 16), F32),
        pltpu.VMEM((RL, 16), F32),
        pltpu.VMEM((RL,), I32),
        pltpu.VMEM((RL,), I32),
        pltpu.VMEM_SHARED((ACC_ROWS, 16), F32),
        pltpu.SemaphoreType.DMA,
        pltpu.SemaphoreType.DMA,
    ],
)
def _sc_stage_b(ubar_hbm, nm_hbm, node_hbm, hedge_hbm, z_hbm, out_hbm,
                nmv, nodeb, hedgeb, ur0, ur1, sl0, sl1, accS, sg0, sg1):
    c = lax.axis_index("c")
    s = lax.axis_index("s")
    wid = s * NC + c
    rowbase = wid * RW
    zrows = ACC_ROWS // NS

    pltpu.sync_copy(z_hbm.at[pl.ds(s * zrows, zrows)],
                    accS.at[pl.ds(s * zrows, zrows)])
    pltpu.sync_copy(nm_hbm, nmv)
    pltpu.sync_copy(node_hbm.at[pl.ds(rowbase, RW)], nodeb)
    pltpu.sync_copy(hedge_hbm.at[pl.ds(rowbase, RW)], hedgeb)
    plsc.subcore_barrier()

    def issue(j, ur, sg, slotb):
        pltpu.async_copy(ubar_hbm.at[hedgeb.at[j]], ur, sg)

        def inner(i, icarry):
            idx = nodeb[j, pl.ds(i * 16, 16)]
            sl = plsc.load_gather(nmv, [idx])
            slotb[pl.ds(i * 16, 16)] = jnp.where(
                sl == M, M + jnp.bitwise_and(idx, TRASH - 1), sl)
            return icarry

        lax.fori_loop(0, RL // 16, inner, 0)

    def drain(j, ur, sg, slotb):
        pltpu.make_async_copy(ubar_hbm.at[hedgeb.at[j]], ur, sg).wait()
        pltpu.sync_copy(ur, accS.at[slotb], add=True)

    issue(0, ur0, sg0, sl0)

    def body(jj, carry):
        j0 = jj * 2
        j1 = j0 + 1
        issue(j1, ur1, sg1, sl1)
        drain(j0, ur0, sg0, sl0)

        @pl.when(jj < RW // 2 - 1)
        def _():
            issue(j0 + 2, ur0, sg0, sl0)

        drain(j1, ur1, sg1, sl1)
        return carry

    lax.fori_loop(0, RW // 2, body, 0)
    plsc.subcore_barrier()

    @pl.when(s < 13)
    def _():
        pltpu.sync_copy(accS.at[pl.ds(s * 40, 40)],
                        out_hbm.at[c, pl.ds(s * 40, 40)])


def _tc_macro_body(acc_ref, ms_ref, mg_ref,
                   W1_ref, b1_ref, Wp_ref, W2l_ref, b2l_ref, W2r_ref,
                   W3l_ref, b3l_ref, W3r_ref, Wm1_ref, bm1_ref,
                   Wm2_ref, bm2_ref, Wm3_ref, bm3_ref, out_ref):
    hi = lax.Precision.HIGHEST
    acc = acc_ref[0] + acc_ref[1]
    cols = lax.broadcasted_iota(I32, (M, MROWS), 1)
    onehot = (cols == ms_ref[...]).astype(F32)
    H = jnp.dot(onehot, acc, precision=hi)
    U5, U4 = H[:, 0:5], H[:, 5:9]
    T, C = H[:, 9:10], H[:, 10:11]
    pre = (jnp.dot(U5, W1_ref[...], precision=hi) + T * b1_ref[...]
           + jnp.dot(U4, Wp_ref[...], precision=hi))
    hm = _leaky(pre / jnp.maximum(C, 1.0))

    mg = mg_ref[...]
    px, py, sx, sy = mg[:, 0:1], mg[:, 1:2], mg[:, 2:3], mg[:, 3:4]
    ri = lax.broadcasted_iota(I32, (M, M), 0)
    ci = lax.broadcasted_iota(I32, (M, M), 1)
    eye = (ri == ci).astype(F32)
    noteye = 1.0 - eye
    dn = (((0,), (0,)), ((), ()))
    pxr = lax.dot_general(px, eye, dn, precision=hi)
    pyr = lax.dot_general(py, eye, dn, precision=hi)
    adj_v = ((px + sx) <= pxr).astype(F32) * noteye
    adj_h = ((py + sy) <= pyr).astype(F32) * noteye
    ones_col = jnp.ones((M, 1), F32)

    def sage(xf, adj, Wl, bl, Wr):
        deg = lax.dot_general(adj, ones_col, dn, precision=hi)
        agg = lax.dot_general(adj, xf, dn)
        return jnp.dot(agg / jnp.maximum(deg, 1.0), Wl) + bl + jnp.dot(xf, Wr)

    xv = _leaky(sage(hm, adj_v, W2l_ref[...], b2l_ref[...], W2r_ref[...]))
    xh = _leaky(sage(hm, adj_h, W2l_ref[...], b2l_ref[...], W2r_ref[...]))
    xc = jnp.concatenate([xv, xh], axis=1)
    xv2 = _leaky(sage(xc, adj_v, W3l_ref[...], b3l_ref[...], W3r_ref[...]))
    xh2 = _leaky(sage(xc, adj_h, W3l_ref[...], b3l_ref[...], W3r_ref[...]))
    xc2 = jnp.concatenate([xv2, xh2], axis=1)
    o = _leaky(jnp.dot(xc2, Wm1_ref[...]) + bm1_ref[...])
    o = _leaky(jnp.dot(o, Wm2_ref[...]) + bm2_ref[...])
    out_ref[...] = jnp.dot(o, Wm3_ref[...]) + bm3_ref[...]


_tc_macro = pl.pallas_call(
    _tc_macro_body, out_shape=jax.ShapeDtypeStruct((M, 4), F32))


def kernel(x, edge_index, pin_feature, batch, fake_pos, macro_index,
           W1, b1, Wp, W2l, b2l, W2r, W3l, b3l, W3r,
           Wm1, bm1, Wm2, bm2, Wm3, bm3):
    del batch
    node_idx = edge_index[0]
    hedge_idx = edge_index[1]

    pad = PP - P
    prng = jnp.arange(pad, dtype=I32)
    node_pa = jnp.concatenate([node_idx, prng % N]).reshape(NW * RW, RL)
    node_pb = jnp.concatenate([node_idx, N + (prng % 16)]).reshape(NW * RW, RL)
    hedge_p = jnp.concatenate([hedge_idx, NH + (prng % 16)]).reshape(NW * RW, RL)
    pfw = jnp.concatenate(
        [jnp.zeros((P, 5), F32), pin_feature, jnp.zeros((P, 7), F32)], axis=1)
    pfw = jnp.concatenate([pfw, jnp.zeros((pad, 16), F32)], axis=0)
    x_p = jnp.concatenate([x, jnp.zeros((NP - N, 2), F32)], axis=0)
    fp_p = jnp.concatenate([fake_pos, jnp.zeros((NP - N, 2), F32)], axis=0)
    zeros16 = jnp.zeros((NP, 16), F32)

    nodemap, macroslot, mgeo = _sc_prep(macro_index, x_p.reshape(-1), fp_p.reshape(-1))
    xin16 = _tc_xin(x_p, fp_p, nodemap.reshape(NP, 1))
    accA = _sc_stage_a(xin16, pfw, node_pa, hedge_p, zeros16)
    ubar = _tc_ubar(accA)
    accB = _sc_stage_b(ubar, nodemap, node_pb, hedge_p, zeros16)
    out = _tc_macro(accB, macroslot.reshape(M, 1), mgeo,
                    W1, b1.reshape(1, 128), Wp, W2l, b2l.reshape(1, 128), W2r,
                    W3l, b3l.reshape(1, 128), W3r, Wm1, bm1.reshape(1, 128),
                    Wm2, bm2.reshape(1, 64), Wm3, bm3.reshape(1, 4))
    return out
